# Initial kernel scaffold; baseline (speedup 1.0000x reference)
import jax
import jax.numpy as jnp
from jax.experimental import pallas as pl


def kernel(x, edge_index, Wl1, Wr1, b1, Wl2, Wr2, b2, Wl3, Wr3, b3):
    raise NotImplementedError("write your pallas kernel here")


# speedup vs baseline: 3.5709x; 3.5709x over previous
import functools

import jax
import jax.numpy as jnp
from jax import lax
from jax.experimental import pallas as pl
from jax.experimental.pallas import tpu as pltpu
from jax.experimental.pallas import tpu_sc as plsc

NC = 2
NS = 16
NW = NC * NS
CH = 128
ZB = 128


def _make_sc_aggregate(n, d, k, n_pad, nchunks):
    mesh = plsc.VectorSubcoreMesh(core_axis_name="c", subcore_axis_name="s",
                                  num_cores=NC, num_subcores=NS)
    nps = n_pad // NS
    zsteps = n_pad // (NS * ZB)

    def body(feats_hbm, src_hbm, dst_hbm, zd_hbm, out_hbm,
             src_v, dst_v, rows_v, agg_sh, sem):
        c = lax.axis_index("c")
        s = lax.axis_index("s")
        w = c * NS + s

        pltpu.sync_copy(zd_hbm, rows_v)
        for t in range(zsteps):
            pltpu.sync_copy(rows_v, agg_sh.at[pl.ds(s * nps + t * ZB, ZB)])
        plsc.subcore_barrier()

        def chunk(j, carry):
            pltpu.sync_copy(src_hbm.at[w, j], src_v)
            pltpu.sync_copy(dst_hbm.at[w, j], dst_v)
            pltpu.async_copy(feats_hbm.at[src_v], rows_v, sem).wait()
            pltpu.sync_copy(rows_v, agg_sh.at[dst_v], add=True)
            return carry

        lax.fori_loop(0, nchunks, chunk, 0)
        plsc.subcore_barrier()

        for t in range(zsteps):
            r0 = s * nps + t * ZB
            pltpu.sync_copy(agg_sh.at[pl.ds(r0, ZB)], rows_v)
            pltpu.sync_copy(rows_v, out_hbm.at[c, pl.ds(r0, ZB)])

    return pl.kernel(
        body,
        out_type=jax.ShapeDtypeStruct((NC, n_pad, d), jnp.float32),
        mesh=mesh,
        scratch_types=[
            pltpu.VMEM((CH,), jnp.int32),
            pltpu.VMEM((CH,), jnp.int32),
            pltpu.VMEM((CH, d), jnp.float32),
            pltpu.VMEM_SHARED((n_pad, d), jnp.float32),
            pltpu.SemaphoreType.DMA,
        ])


def _make_sc_degree(n, d, k, n_pad, nchunks):
    mesh = plsc.VectorSubcoreMesh(core_axis_name="c", subcore_axis_name="s",
                                  num_cores=NC, num_subcores=NS)
    nps = n_pad // NS
    zsteps = n_pad // (NS * ZB)

    def body(dst_hbm, zd_hbm, ones_hbm, cnt_hbm, dst_v, ones_v, cnt_sh):
        c = lax.axis_index("c")
        s = lax.axis_index("s")
        w = c * NS + s

        pltpu.sync_copy(zd_hbm, ones_v)
        for t in range(zsteps):
            pltpu.sync_copy(ones_v, cnt_sh.at[pl.ds(s * nps + t * ZB, ZB)])
        pltpu.sync_copy(ones_hbm, ones_v)
        plsc.subcore_barrier()

        def chunk(j, carry):
            pltpu.sync_copy(dst_hbm.at[w, j], dst_v)
            pltpu.sync_copy(ones_v, cnt_sh.at[dst_v], add=True)
            return carry

        lax.fori_loop(0, nchunks, chunk, 0)
        plsc.subcore_barrier()

        for t in range(zsteps):
            r0 = s * nps + t * ZB
            pltpu.sync_copy(cnt_sh.at[pl.ds(r0, ZB)], ones_v)
            pltpu.sync_copy(ones_v, cnt_hbm.at[c, pl.ds(r0, ZB)])

    return pl.kernel(
        body,
        out_type=jax.ShapeDtypeStruct((NC, n_pad, d), jnp.float32),
        mesh=mesh,
        scratch_types=[
            pltpu.VMEM((CH,), jnp.int32),
            pltpu.VMEM((CH, d), jnp.float32),
            pltpu.VMEM_SHARED((n_pad, d), jnp.float32),
        ])


def _comb_body(a_ref, c_ref, h_ref, wl_ref, wr_ref, b_ref, o_ref, *, last):
    cnt = c_ref[0, :, 0:1] + c_ref[1, :, 0:1]
    inv = 1.0 / jnp.maximum(cnt, 1.0)
    mean = (a_ref[0] + a_ref[1]) * inv
    z = (jnp.dot(mean, wl_ref[...], preferred_element_type=jnp.float32)
         + jnp.dot(h_ref[...], wr_ref[...], preferred_element_type=jnp.float32)
         + b_ref[...])
    z = jnp.where(z > 0, z, jnp.exp(jnp.minimum(z, 0.0)) - 1.0)
    if last:
        m = jnp.max(z, axis=1, keepdims=True)
        z = z - m - jnp.log(jnp.sum(jnp.exp(z - m), axis=1, keepdims=True))
    o_ref[...] = z


def _tc_layer(agg, cnt, h, wl, wr, b, *, blk, last):
    n, d = h.shape
    grid = (n // blk,)
    return pl.pallas_call(
        functools.partial(_comb_body, last=last),
        grid=grid,
        in_specs=[
            pl.BlockSpec((NC, blk, d), lambda i: (0, i, 0)),
            pl.BlockSpec((NC, blk, d), lambda i: (0, i, 0)),
            pl.BlockSpec((blk, d), lambda i: (i, 0)),
            pl.BlockSpec((d, d), lambda i: (0, 0)),
            pl.BlockSpec((d, d), lambda i: (0, 0)),
            pl.BlockSpec((1, d), lambda i: (0, 0)),
        ],
        out_specs=pl.BlockSpec((blk, d), lambda i: (i, 0)),
        out_shape=jax.ShapeDtypeStruct((n, d), jnp.float32),
    )(agg, cnt, h, wl, wr, b.reshape(1, d))


def kernel(x, edge_index, Wl1, Wr1, b1, Wl2, Wr2, b2, Wl3, Wr3, b3):
    n, d = x.shape
    e = edge_index.shape[1]
    assert n % NS == 0 and d == 128

    k = -(-e // (NW * CH))
    e_pad = NW * CH * k
    n_pad = -(-(n + 1) // (NS * ZB)) * NS * ZB

    src = edge_index[0].astype(jnp.int32)
    dst = edge_index[1].astype(jnp.int32)
    src3 = jnp.pad(src, (0, e_pad - e)).reshape(NW, k, CH)
    dst3 = jnp.pad(dst, (0, e_pad - e),
                   constant_values=n).reshape(NW, k, CH)

    zeros_d = jnp.zeros((ZB, d), jnp.float32)
    ones_d = jnp.ones((CH, d), jnp.float32)

    sc_agg = _make_sc_aggregate(n, d, k, n_pad, k)
    sc_deg = _make_sc_degree(n, d, k, n_pad, k)

    cnt = sc_deg(dst3, zeros_d, ones_d)
    a1 = sc_agg(x, src3, dst3, zeros_d)
    h1 = _tc_layer(a1, cnt, x, Wl1, Wr1, b1, blk=1000, last=False)
    a2 = sc_agg(h1, src3, dst3, zeros_d)
    h2 = _tc_layer(a2, cnt, h1, Wl2, Wr2, b2, blk=1000, last=False)
    a3 = sc_agg(h2, src3, dst3, zeros_d)
    return _tc_layer(a3, cnt, h2, Wl3, Wr3, b3, blk=1000, last=True)
